# baseline (speedup 1.0000x reference)
import functools

import jax
import jax.numpy as jnp
from jax import lax
from jax.experimental import pallas as pl
from jax.experimental.pallas import tpu as pltpu
from jax.experimental.pallas import tpu_sc as plsc

_LANES = 16


def _build_gather(B, C, K, D, chunk):
    info = plsc.get_sparse_core_info()
    nc, ns = info.num_cores, info.num_subcores
    nw = nc * ns
    rows_out = B * K
    rows_per_worker = rows_out // nw
    n_chunks = rows_per_worker // chunk
    assert rows_out % nw == 0 and rows_per_worker % chunk == 0
    assert rows_per_worker == K and B == nw

    mesh = plsc.VectorSubcoreMesh(core_axis_name="c", subcore_axis_name="s")

    @functools.partial(
        pl.kernel,
        mesh=mesh,
        out_type=jax.ShapeDtypeStruct((rows_out, D), jnp.float32),
        compiler_params=pltpu.CompilerParams(use_tc_tiling_on_sc=False),
        scratch_types=[
            pltpu.VMEM((K,), jnp.int32),
            pltpu.VMEM((n_chunks, chunk), jnp.int32),
            pltpu.VMEM((2, chunk, D), jnp.float32),
            pltpu.SemaphoreType.DMA,
            pltpu.SemaphoreType.DMA,
        ],
    )
    def gather_rows(x_hbm, keep_hbm, out_hbm, keep_v, idx_v, bufs, gsem, ssem):
        wid = lax.axis_index("s") * nc + lax.axis_index("c")
        base_row = wid * C
        out_base = wid * K

        pltpu.sync_copy(keep_hbm, keep_v)
        for k in range(n_chunks):
            for i in range(chunk // _LANES):
                s = pl.ds(k * chunk + i * _LANES, _LANES)
                idx_v[k, pl.ds(i * _LANES, _LANES)] = keep_v[s] + base_row

        def start_gather(k):
            return pltpu.async_copy(
                x_hbm.at[idx_v.at[k]], bufs.at[k % 2], gsem
            )

        def start_scatter(k):
            return pltpu.async_copy(
                bufs.at[k % 2],
                out_hbm.at[pl.ds(out_base + k * chunk, chunk)],
                ssem,
            )

        scatters = [None, None]
        gathers = [None] * n_chunks
        gathers[0] = start_gather(0)
        for k in range(n_chunks):
            if k + 1 < n_chunks:
                if scatters[(k + 1) % 2] is not None:
                    scatters[(k + 1) % 2].wait()
                gathers[k + 1] = start_gather(k + 1)
            gathers[k].wait()
            scatters[k % 2] = start_scatter(k)
        for h in scatters:
            if h is not None:
                h.wait()

    return gather_rows


def kernel(x, keep):
    B, C, H, W = x.shape
    K = keep.shape[0]
    D = H * W
    x_flat = x.reshape(B * C, D)
    gather_rows = _build_gather(B, C, K, D, chunk=16)
    out = gather_rows(x_flat, keep)
    return out.reshape(B, K, H, W)
